# baseline (speedup 1.0000x reference)
import functools

import jax
import jax.numpy as jnp
from jax import lax
from jax.experimental import pallas as pl
from jax.experimental.pallas import tpu as pltpu, tpu_sc as plsc

D_MODEL = 1024
NUM_POSITIONS = 8
MAX_NODE = 15
BATCH = 16384
UNIT_D = D_MODEL // NUM_POSITIONS

NW = 32
CH = 128
BPW = BATCH // NW
CPW = BPW // CH
NB = 7
NCHUNK = CPW * NUM_POSITIONS
PAD_NODE = 16
NREP = 8


def _sc_gather(idx, tab16):
    mesh = plsc.VectorSubcoreMesh(core_axis_name="c", subcore_axis_name="s")

    @functools.partial(
        pl.kernel,
        out_type=jax.ShapeDtypeStruct((BATCH, D_MODEL), jnp.float32),
        mesh=mesh,
        scratch_types=[
            pltpu.VMEM((NUM_POSITIONS, BPW), jnp.int32),
            pltpu.VMEM((NB, CH, UNIT_D), jnp.float32),
            pltpu.VMEM_SHARED((NREP, NUM_POSITIONS, PAD_NODE, UNIT_D),
                              jnp.float32),
            pltpu.SemaphoreType.DMA((NB,)),
            pltpu.SemaphoreType.DMA((NB,)),
        ],
    )
    def k(idx_hbm, tab_hbm, out_hbm, idx_v, rows_v, tab_sp, gsem, ssem):
        sid = lax.axis_index("s")
        wid = sid * 2 + lax.axis_index("c")
        b0 = wid * BPW
        rep = lax.rem(sid, NREP)

        @pl.when(sid < NREP)
        def _copy_table():
            pltpu.sync_copy(tab_hbm, tab_sp.at[sid])

        pltpu.sync_copy(idx_hbm.at[:, pl.ds(b0, BPW)], idx_v)
        plsc.subcore_barrier()

        def gather_d(j):
            i = lax.rem(j, NUM_POSITIONS)
            cb = lax.div(j, NUM_POSITIONS)
            p = lax.rem(j, NB)
            return pltpu.make_async_copy(
                tab_sp.at[rep, i].at[idx_v.at[i, pl.ds(cb * CH, CH)]],
                rows_v.at[p], gsem.at[p])

        def store_d(j):
            i = lax.rem(j, NUM_POSITIONS)
            cb = lax.div(j, NUM_POSITIONS)
            p = lax.rem(j, NB)
            return pltpu.make_async_copy(
                rows_v.at[p],
                out_hbm.at[pl.ds(pl.multiple_of(b0 + cb * CH, CH), CH),
                           pl.ds(pl.multiple_of(i * UNIT_D, UNIT_D), UNIT_D)],
                ssem.at[p])

        LOOKAHEAD = NB - 2
        for b in range(LOOKAHEAD):
            gather_d(jnp.int32(b)).start()

        def body(j):
            gather_d(j).wait()
            store_d(j).start()
            jn = j + LOOKAHEAD

            @pl.when(jn < NCHUNK)
            def _prefetch():
                @pl.when(jn >= NB)
                def _drain():
                    store_d(jn - NB).wait()
                gather_d(jn).start()

        pl.loop(0, NCHUNK)(body)
        for b in range(NB):
            store_d(jnp.int32(NCHUNK - NB + b)).wait()

    return k(idx, tab16)


def kernel(positional_indices, tables):
    idx = positional_indices.astype(jnp.int32)
    tab16 = jnp.pad(tables, ((0, 0), (0, PAD_NODE - MAX_NODE), (0, 0)))
    return _sc_gather(idx, tab16)


# speedup vs baseline: 1.0114x; 1.0114x over previous
import functools

import jax
import jax.numpy as jnp
from jax import lax
from jax.experimental import pallas as pl
from jax.experimental.pallas import tpu as pltpu, tpu_sc as plsc

D_MODEL = 1024
NUM_POSITIONS = 8
MAX_NODE = 15
BATCH = 16384
UNIT_D = D_MODEL // NUM_POSITIONS

NW = 32
CH = 128
BPW = BATCH // NW
CPW = BPW // CH
NB = 7
NCHUNK = CPW * NUM_POSITIONS
PAD_NODE = 16


def _sc_gather(idx, tab16):
    mesh = plsc.VectorSubcoreMesh(core_axis_name="c", subcore_axis_name="s")

    @functools.partial(
        pl.kernel,
        out_type=jax.ShapeDtypeStruct((BATCH, D_MODEL), jnp.float32),
        mesh=mesh,
        scratch_types=[
            pltpu.VMEM((NUM_POSITIONS, BPW), jnp.int32),
            pltpu.VMEM((NB, CH, UNIT_D), jnp.float32),
            pltpu.VMEM_SHARED((NUM_POSITIONS, PAD_NODE, UNIT_D), jnp.float32),
            pltpu.SemaphoreType.DMA((NB,)),
            pltpu.SemaphoreType.DMA((NB,)),
        ],
    )
    def k(idx_hbm, tab_hbm, out_hbm, idx_v, rows_v, tab_sp, gsem, ssem):
        wid = lax.axis_index("s") * 2 + lax.axis_index("c")
        b0 = wid * BPW

        @pl.when(lax.axis_index("s") == 0)
        def _copy_table():
            pltpu.sync_copy(tab_hbm, tab_sp)

        pltpu.sync_copy(idx_hbm.at[:, pl.ds(b0, BPW)], idx_v)
        plsc.subcore_barrier()

        def gather_d(j):
            i = lax.rem(j, NUM_POSITIONS)
            cb = lax.div(j, NUM_POSITIONS)
            p = lax.rem(j, NB)
            return pltpu.make_async_copy(
                tab_sp.at[i].at[idx_v.at[i, pl.ds(cb * CH, CH)]],
                rows_v.at[p], gsem.at[p])

        def store_d(j):
            i = lax.rem(j, NUM_POSITIONS)
            cb = lax.div(j, NUM_POSITIONS)
            p = lax.rem(j, NB)
            return pltpu.make_async_copy(
                rows_v.at[p],
                out_hbm.at[pl.ds(pl.multiple_of(b0 + cb * CH, CH), CH),
                           pl.ds(pl.multiple_of(i * UNIT_D, UNIT_D), UNIT_D)],
                ssem.at[p])

        LOOKAHEAD = NB - 2
        for b in range(LOOKAHEAD):
            gather_d(jnp.int32(b)).start()

        def body(j):
            jn = j + LOOKAHEAD

            @pl.when(jnp.logical_and(jn < NCHUNK, jn >= NB))
            def _drain():
                store_d(jn - NB).wait()

            gather_d(j).wait()
            store_d(j).start()

            @pl.when(jn < NCHUNK)
            def _prefetch():
                gather_d(jn).start()

        pl.loop(0, NCHUNK)(body)
        for b in range(NB):
            store_d(jnp.int32(NCHUNK - NB + b)).wait()

    return k(idx, tab16)


def kernel(positional_indices, tables):
    idx = positional_indices.astype(jnp.int32)
    tab16 = jnp.pad(tables, ((0, 0), (0, PAD_NODE - MAX_NODE), (0, 0)))
    return _sc_gather(idx, tab16)
